# baseline (speedup 1.0000x reference)
import jax
import jax.numpy as jnp
from jax.experimental import pallas as pl
from jax.experimental.pallas import tpu as pltpu


_BM = 400


def _layer1_body(adj_ref, h_ref, w_ref, out_ref, adj16_ref):
    a16 = adj_ref[...].astype(jnp.bfloat16)
    adj16_ref[...] = a16
    t = jnp.dot(a16, h_ref[...], preferred_element_type=jnp.float32)
    t = jnp.dot(t, w_ref[...], preferred_element_type=jnp.float32)
    out_ref[...] = jnp.maximum(t, 0.0).astype(out_ref.dtype)


def _layer1(adj, h, w):
    m, k = adj.shape
    d = w.shape[1]
    return pl.pallas_call(
        _layer1_body,
        grid=(pl.cdiv(m, _BM),),
        in_specs=[
            pl.BlockSpec((_BM, k), lambda i: (i, 0)),
            pl.BlockSpec((k, d), lambda i: (0, 0)),
            pl.BlockSpec((d, d), lambda i: (0, 0)),
        ],
        out_specs=[
            pl.BlockSpec((_BM, d), lambda i: (i, 0)),
            pl.BlockSpec((_BM, k), lambda i: (i, 0)),
        ],
        out_shape=[
            jax.ShapeDtypeStruct((m, d), jnp.bfloat16),
            jax.ShapeDtypeStruct((m, k), jnp.bfloat16),
        ],
        compiler_params=pltpu.CompilerParams(
            dimension_semantics=("parallel",),
        ),
    )(adj, h, w)


def _layer_body(adj_ref, h_ref, w_ref, out_ref):
    t = jnp.dot(adj_ref[...], h_ref[...], preferred_element_type=jnp.float32)
    t = jnp.dot(t, w_ref[...], preferred_element_type=jnp.float32)
    out_ref[...] = jnp.maximum(t, 0.0).astype(out_ref.dtype)


def _layer(adj, h, w, out_dtype, bm):
    m, k = adj.shape
    d = w.shape[1]
    return pl.pallas_call(
        _layer_body,
        grid=(pl.cdiv(m, bm),),
        in_specs=[
            pl.BlockSpec((bm, k), lambda i: (i, 0)),
            pl.BlockSpec((k, d), lambda i: (0, 0)),
            pl.BlockSpec((d, d), lambda i: (0, 0)),
        ],
        out_specs=pl.BlockSpec((bm, d), lambda i: (i, 0)),
        out_shape=jax.ShapeDtypeStruct((m, d), out_dtype),
        compiler_params=pltpu.CompilerParams(
            dimension_semantics=("parallel",),
        ),
    )(adj, h, w)


def kernel(x, adj, W0, W1, W2):
    h, adj16 = _layer1(adj, x.astype(jnp.bfloat16), W0)
    h = _layer(adj16, h, W1, jnp.bfloat16, 400)
    return _layer(adj16, h, W2, jnp.float32, 400)


# speedup vs baseline: 1.0435x; 1.0435x over previous
import jax
import jax.numpy as jnp
from jax.experimental import pallas as pl


def _layer1_body(adj_ref, h_ref, w_ref, out_ref, adj16_ref):
    a16 = adj_ref[...].astype(jnp.bfloat16)
    adj16_ref[...] = a16
    t = jnp.dot(a16, h_ref[...], preferred_element_type=jnp.float32)
    t = jnp.dot(t, w_ref[...], preferred_element_type=jnp.float32)
    out_ref[...] = jnp.maximum(t, 0.0).astype(out_ref.dtype)


def _layer1(adj, h, w, bm):
    m, k = adj.shape
    d = w.shape[1]
    return pl.pallas_call(
        _layer1_body,
        grid=(pl.cdiv(m, bm),),
        in_specs=[
            pl.BlockSpec((bm, k), lambda i: (i, 0)),
            pl.BlockSpec((k, d), lambda i: (0, 0)),
            pl.BlockSpec((d, d), lambda i: (0, 0)),
        ],
        out_specs=[
            pl.BlockSpec((bm, d), lambda i: (i, 0)),
            pl.BlockSpec((bm, k), lambda i: (i, 0)),
        ],
        out_shape=[
            jax.ShapeDtypeStruct((m, d), jnp.bfloat16),
            jax.ShapeDtypeStruct((m, k), jnp.bfloat16),
        ],
    )(adj, h, w)


def _layer_body(adj_ref, h_ref, w_ref, out_ref):
    t = jnp.dot(adj_ref[...], h_ref[...], preferred_element_type=jnp.float32)
    t = jnp.dot(t, w_ref[...], preferred_element_type=jnp.float32)
    out_ref[...] = jnp.maximum(t, 0.0).astype(out_ref.dtype)


def _layer(adj, h, w, out_dtype, bm):
    m, k = adj.shape
    d = w.shape[1]
    return pl.pallas_call(
        _layer_body,
        grid=(pl.cdiv(m, bm),),
        in_specs=[
            pl.BlockSpec((bm, k), lambda i: (i, 0)),
            pl.BlockSpec((k, d), lambda i: (0, 0)),
            pl.BlockSpec((d, d), lambda i: (0, 0)),
        ],
        out_specs=pl.BlockSpec((bm, d), lambda i: (i, 0)),
        out_shape=jax.ShapeDtypeStruct((m, d), out_dtype),
    )(adj, h, w)


def kernel(x, adj, W0, W1, W2):
    h, adj16 = _layer1(adj, x.astype(jnp.bfloat16), W0, 400)
    h = _layer(adj16, h, W1, jnp.bfloat16, 1000)
    return _layer(adj16, h, W2, jnp.float32, 1000)
